# baseline (speedup 1.0000x reference)
import functools

import jax
import jax.numpy as jnp
from jax import lax
from jax.experimental import pallas as pl
from jax.experimental.pallas import tpu as pltpu
from jax.experimental.pallas import tpu_sc as plsc

D = 64
VPAD = 32
NW = 32
CROWS = 512
L = 16


def _fuse_table_body(emb_ref, pt_ref, wpt_ref, bp_ref, wjt_ref, bj_ref, out_ref):
    prop = jnp.dot(pt_ref[...], wpt_ref[...],
                   preferred_element_type=jnp.float32) + bp_ref[...]
    combined = jnp.concatenate([emb_ref[...], prop], axis=-1)
    out_ref[...] = jnp.dot(combined, wjt_ref[...],
                           preferred_element_type=jnp.float32) + bj_ref[...]


def _build_table(emb_table, property_table, Wp, bp, Wj, bj):
    vocab = emb_table.shape[0]
    return pl.pallas_call(
        _fuse_table_body,
        out_shape=jax.ShapeDtypeStruct((vocab, D), jnp.float32),
    )(emb_table, property_table, Wp.T, bp.reshape(1, D), Wj.T,
      bj.reshape(1, D))


@functools.cache
def _make_gather(n_idx):
    per_w = n_idx // NW
    n_chunks = per_w // CROWS
    n_groups = CROWS // L
    mesh = plsc.VectorSubcoreMesh(core_axis_name="c", subcore_axis_name="s")

    @functools.partial(
        pl.kernel, mesh=mesh,
        compiler_params=pltpu.CompilerParams(
            use_tc_tiling_on_sc=False, needs_layout_passes=False),
        out_type=jax.ShapeDtypeStruct((n_idx * D,), jnp.float32),
        scratch_types=[
            pltpu.VMEM((per_w,), jnp.int32),
            pltpu.VMEM((VPAD * D,), jnp.float32),
            pltpu.VMEM((2, CROWS * D), jnp.float32),
            pltpu.SemaphoreType.DMA,
            pltpu.SemaphoreType.DMA,
        ],
    )
    def gather(table_hbm, idx_hbm, out_hbm, idx_v, table_v, rows_v, o0, o1):
        wid = lax.axis_index("s") * 2 + lax.axis_index("c")
        base = wid * per_w
        pltpu.sync_copy(table_hbm, table_v)
        pltpu.sync_copy(idx_hbm.at[pl.ds(base, per_w)], idx_v)
        osems = (o0, o1)
        lanes = lax.iota(jnp.int32, L)

        def compute_chunk(c, h):
            rows_ref = rows_v.at[h]

            def group(g, carry):
                idx16 = idx_v[pl.ds(c * CROWS + g * L, L)]
                src = idx16 << 6
                dst = (g * L + lanes) << 6
                for j in range(D):
                    v = plsc.load_gather(table_v, [src + j])
                    plsc.store_scatter(rows_ref, [dst + j], v)
                return carry

            lax.fori_loop(0, n_groups, group, 0)

        def o_copy(c, h):
            return pltpu.make_async_copy(
                rows_v.at[h],
                out_hbm.at[pl.ds((base + c * CROWS) * D, CROWS * D)],
                osems[h])

        for c in range(n_chunks):
            h = c % 2
            if c >= 2:
                o_copy(c - 2, h).wait()
            compute_chunk(c, h)
            o_copy(c, h).start()
        o_copy(n_chunks - 2, 0).wait()
        o_copy(n_chunks - 1, 1).wait()

    return gather


def kernel(x, emb_table, Wp, bp, Wj, bj, property_table):
    b, s = x.shape
    n = b * s
    table = _build_table(emb_table, property_table, Wp, bp, Wj, bj)
    table_flat = jnp.pad(table, ((0, VPAD - table.shape[0]), (0, 0))).reshape(-1)
    idx_flat = x.reshape(n).astype(jnp.int32)
    out = _make_gather(n)(table_flat, idx_flat)
    return out.reshape(b, s, D)


# speedup vs baseline: 3.4229x; 3.4229x over previous
import functools

import jax
import jax.numpy as jnp
from jax import lax
from jax.experimental import pallas as pl
from jax.experimental.pallas import tpu as pltpu
from jax.experimental.pallas import tpu_sc as plsc

D = 64
NW = 32
CHUNK = 128


def _fuse_table_body(emb_ref, pt_ref, wpt_ref, bp_ref, wjt_ref, bj_ref, out_ref):
    prop = jnp.dot(pt_ref[...], wpt_ref[...],
                   preferred_element_type=jnp.float32) + bp_ref[...]
    combined = jnp.concatenate([emb_ref[...], prop], axis=-1)
    out_ref[...] = jnp.dot(combined, wjt_ref[...],
                           preferred_element_type=jnp.float32) + bj_ref[...]


def _build_table(emb_table, property_table, Wp, bp, Wj, bj):
    vocab = emb_table.shape[0]
    return pl.pallas_call(
        _fuse_table_body,
        out_shape=jax.ShapeDtypeStruct((vocab, D), jnp.float32),
    )(emb_table, property_table, Wp.T, bp.reshape(1, D), Wj.T,
      bj.reshape(1, D))


K = 4


@functools.cache
def _make_gather(n_idx):
    per_w = n_idx // NW
    n_chunks = per_w // CHUNK
    n_phases = n_chunks // K
    mesh = plsc.VectorSubcoreMesh(core_axis_name="c", subcore_axis_name="s")

    @functools.partial(
        pl.kernel, mesh=mesh,
        compiler_params=pltpu.CompilerParams(use_tc_tiling_on_sc=False),
        out_type=jax.ShapeDtypeStruct((n_idx, D), jnp.float32),
        scratch_types=[
            pltpu.VMEM((n_chunks, CHUNK), jnp.int32),
            pltpu.VMEM((2, K, CHUNK, D), jnp.float32),
            pltpu.VMEM_SHARED((32, D), jnp.float32),
            pltpu.SemaphoreType.DMA,
            pltpu.SemaphoreType.DMA,
            pltpu.SemaphoreType.DMA,
            pltpu.SemaphoreType.DMA,
        ],
    )
    def gather(table_hbm, idx_hbm, out_hbm, idx_v, rows_v, table_sh,
               g0, g1, o0, o1):
        wid = lax.axis_index("s") * 2 + lax.axis_index("c")
        base = wid * per_w
        @pl.when(lax.axis_index("s") == 0)
        def _():
            pltpu.sync_copy(table_hbm, table_sh.at[pl.ds(0, 25), :])

        pltpu.sync_copy(idx_hbm.at[pl.ds(wid * n_chunks, n_chunks), :], idx_v)
        plsc.subcore_barrier()
        gsems = (g0, g1)
        osems = (o0, o1)

        def g_copy(p, h, c):
            j = p * K + c
            return pltpu.make_async_copy(
                table_sh.at[idx_v.at[j]], rows_v.at[h].at[c], gsems[h])

        def o_copy(p, h, c):
            j = p * K + c
            return pltpu.make_async_copy(
                rows_v.at[h].at[c],
                out_hbm.at[pl.ds(base + j * CHUNK, CHUNK), :], osems[h])

        def fire_g(p, h):
            for c in range(K):
                g_copy(p, h, c).start()

        def wait_g(p, h):
            for c in range(K):
                g_copy(p, h, c).wait()

        def fire_o(p, h):
            for c in range(K):
                o_copy(p, h, c).start()

        def wait_o(p, h):
            for c in range(K):
                o_copy(p, h, c).wait()

        fire_g(0, 0)
        wait_g(0, 0)
        fire_o(0, 0)
        fire_g(1, 1)

        def body(i, carry):
            p0 = 2 * i + 1
            wait_g(p0, 1)
            fire_o(p0, 1)
            wait_o(p0 - 1, 0)
            fire_g(p0 + 1, 0)
            p1 = p0 + 1
            wait_g(p1, 0)
            fire_o(p1, 0)
            wait_o(p1 - 1, 1)
            fire_g(p1 + 1, 1)
            return carry

        lax.fori_loop(0, (n_phases - 2) // 2, body, 0)

        p = n_phases - 1
        wait_g(p, 1)
        fire_o(p, 1)
        wait_o(p - 1, 0)
        wait_o(p, 1)

    return gather


def kernel(x, emb_table, Wp, bp, Wj, bj, property_table):
    b, s = x.shape
    n = b * s
    table = _build_table(emb_table, property_table, Wp, bp, Wj, bj)
    idx2d = x.reshape(n // CHUNK, CHUNK).astype(jnp.int32)
    out = _make_gather(n)(table, idx2d)
    return out.reshape(b, s, D)


# speedup vs baseline: 3.4267x; 1.0011x over previous
import functools

import jax
import jax.numpy as jnp
from jax import lax
from jax.experimental import pallas as pl
from jax.experimental.pallas import tpu as pltpu
from jax.experimental.pallas import tpu_sc as plsc

D = 64
NW = 32
CHUNK = 128
K = 4


def _fuse_table_body(emb_ref, pt_ref, wpt_ref, bp_ref, wjt_ref, bj_ref, out_ref):
    prop = jnp.dot(pt_ref[...], wpt_ref[...],
                   preferred_element_type=jnp.float32) + bp_ref[...]
    combined = jnp.concatenate([emb_ref[...], prop], axis=-1)
    out_ref[...] = jnp.dot(combined, wjt_ref[...],
                           preferred_element_type=jnp.float32) + bj_ref[...]


def _build_table(emb_table, property_table, Wp, bp, Wj, bj):
    vocab = emb_table.shape[0]
    return pl.pallas_call(
        _fuse_table_body,
        out_shape=jax.ShapeDtypeStruct((vocab, D), jnp.float32),
    )(emb_table, property_table, Wp.T, bp.reshape(1, D), Wj.T,
      bj.reshape(1, D))


@functools.cache
def _make_gather(b, s):
    n_idx = b * s
    per_w = n_idx // NW
    n_chunks = per_w // CHUNK
    n_phases = n_chunks // K
    rows_per_b = s // CHUNK
    b_per_w = n_chunks // rows_per_b
    mesh = plsc.VectorSubcoreMesh(core_axis_name="c", subcore_axis_name="s")

    @functools.partial(
        pl.kernel, mesh=mesh,
        compiler_params=pltpu.CompilerParams(use_tc_tiling_on_sc=False),
        out_type=jax.ShapeDtypeStruct((b, s, D), jnp.float32),
        scratch_types=[
            pltpu.VMEM((n_chunks, CHUNK), jnp.int32),
            pltpu.VMEM((2, K, CHUNK, D), jnp.float32),
            pltpu.VMEM_SHARED((32, D), jnp.float32),
            pltpu.SemaphoreType.DMA,
            pltpu.SemaphoreType.DMA,
            pltpu.SemaphoreType.DMA,
            pltpu.SemaphoreType.DMA,
        ],
    )
    def gather(table_hbm, idx_hbm, out_hbm, idx_v, rows_v, table_sh,
               g0, g1, o0, o1):
        wid = lax.axis_index("s") * 2 + lax.axis_index("c")
        @pl.when(lax.axis_index("s") == 0)
        def _():
            pltpu.sync_copy(table_hbm, table_sh.at[pl.ds(0, 25), :])

        pltpu.sync_copy(idx_hbm.at[pl.ds(wid * n_chunks, n_chunks), :], idx_v)
        plsc.subcore_barrier()
        gsems = (g0, g1)
        osems = (o0, o1)
        b0 = wid * b_per_w

        def g_copy(p, h, c):
            j = p * K + c
            return pltpu.make_async_copy(
                table_sh.at[idx_v.at[j]], rows_v.at[h].at[c], gsems[h])

        def o_copy(p, h, c):
            j = p * K + c
            return pltpu.make_async_copy(
                rows_v.at[h].at[c],
                out_hbm.at[b0 + j // rows_per_b,
                           pl.ds((j % rows_per_b) * CHUNK, CHUNK), :],
                osems[h])

        def fire_g(p, h):
            for c in range(K):
                g_copy(p, h, c).start()

        def wait_g(p, h):
            for c in range(K):
                g_copy(p, h, c).wait()

        def fire_o(p, h):
            for c in range(K):
                o_copy(p, h, c).start()

        def wait_o(p, h):
            for c in range(K):
                o_copy(p, h, c).wait()

        fire_g(0, 0)
        wait_g(0, 0)
        fire_o(0, 0)
        fire_g(1, 1)

        def body(i, carry):
            p0 = 2 * i + 1
            wait_g(p0, 1)
            fire_o(p0, 1)
            wait_o(p0 - 1, 0)
            fire_g(p0 + 1, 0)
            p1 = p0 + 1
            wait_g(p1, 0)
            fire_o(p1, 0)
            wait_o(p1 - 1, 1)
            fire_g(p1 + 1, 1)
            return carry

        lax.fori_loop(0, (n_phases - 2) // 2, body, 0)

        p = n_phases - 1
        wait_g(p, 1)
        fire_o(p, 1)
        wait_o(p - 1, 0)
        wait_o(p, 1)

    return gather


def kernel(x, emb_table, Wp, bp, Wj, bj, property_table):
    b, s = x.shape
    table = _build_table(emb_table, property_table, Wp, bp, Wj, bj)
    idx2d = x.reshape((b * s) // CHUNK, CHUNK).astype(jnp.int32)
    return _make_gather(b, s)(table, idx2d)
